# baseline (speedup 1.0000x reference)
import jax
import jax.numpy as jnp
from jax import lax
from jax.experimental import pallas as pl
from jax.experimental.pallas import tpu as pltpu
from jax.experimental.pallas import tpu_sc as plsc

N = 10000
N_PAD = 10240
E_TOT = 320000 + N
D = 128
H1 = 8
C2 = 40
C2P = 48
R1 = 144

NC = 2
NS = 16
NW = NC * NS
RPT = N_PAD // NS
N_ACC = 10008
RPA = 632
RPA_LAST = N_ACC - 15 * RPA

K1 = 64
CH1 = 163
EPW1 = K1 * CH1
E_PAD = NW * EPW1
E_ALL = E_PAD + K1

K2 = 128
CH2 = 81
EPW2 = K2 * CH2

_F32 = jnp.float32
_SC_PARAMS = pltpu.CompilerParams(use_tc_tiling_on_sc=False)


def _node1_body(x_ref, w_ref, as_ref, ad_ref, h_ref, s_ref, d_ref):
    h = jnp.dot(x_ref[...], w_ref[...], preferred_element_type=_F32)
    h_ref[...] = h
    s_ref[...] = jnp.dot(h, as_ref[...], preferred_element_type=_F32)
    d_ref[...] = jnp.dot(h, ad_ref[...], preferred_element_type=_F32)


def _node1(xp, W1, As, Ad):
    blk = 512
    return pl.pallas_call(
        _node1_body,
        grid=(N_PAD // blk,),
        in_specs=[
            pl.BlockSpec((blk, D), lambda i: (i, 0)),
            pl.BlockSpec((D, D), lambda i: (0, 0)),
            pl.BlockSpec((D, 16), lambda i: (0, 0)),
            pl.BlockSpec((D, 16), lambda i: (0, 0)),
        ],
        out_specs=[
            pl.BlockSpec((blk, D), lambda i: (i, 0)),
            pl.BlockSpec((blk, 16), lambda i: (i, 0)),
            pl.BlockSpec((blk, 16), lambda i: (i, 0)),
        ],
        out_shape=[
            jax.ShapeDtypeStruct((N_PAD, D), _F32),
            jax.ShapeDtypeStruct((N_PAD, 16), _F32),
            jax.ShapeDtypeStruct((N_PAD, 16), _F32),
        ],
    )(xp, W1, As, Ad)


def _node2_body(m0_ref, m1_ref, e8_ref, b1_ref, w2_ref, h2_ref, d_ref):
    m = m0_ref[...] + m1_ref[...]
    den = jnp.dot(m[:, 128:136], e8_ref[...], preferred_element_type=_F32)
    hr = jnp.maximum(m[:, 0:128] / (den + 1e-16) + b1_ref[...], 0.0)
    h2 = jnp.dot(hr, w2_ref[...], preferred_element_type=_F32)
    h2_ref[...] = h2
    d_ref[...] = jnp.broadcast_to(h2[:, 41:42], (h2.shape[0], 16))


def _node2(msg, E8, b1r, W2e):
    blk = 512
    nb = N_PAD // blk
    return pl.pallas_call(
        _node2_body,
        grid=(nb,),
        in_specs=[
            pl.BlockSpec((blk, R1), lambda i: (i, 0)),
            pl.BlockSpec((blk, R1), lambda i: (i + nb, 0)),
            pl.BlockSpec((8, D), lambda i: (0, 0)),
            pl.BlockSpec((1, D), lambda i: (0, 0)),
            pl.BlockSpec((D, C2P), lambda i: (0, 0)),
        ],
        out_specs=[
            pl.BlockSpec((blk, C2P), lambda i: (i, 0)),
            pl.BlockSpec((blk, 16), lambda i: (i, 0)),
        ],
        out_shape=[
            jax.ShapeDtypeStruct((N_PAD, C2P), _F32),
            jax.ShapeDtypeStruct((N_PAD, 16), _F32),
        ],
    )(msg, msg, E8, b1r, W2e)


def _final_body(m0_ref, m1_ref, b2_ref, o_ref):
    m = m0_ref[...] + m1_ref[...]
    lg = m[:, 0:40] / (m[:, 40:41] + 1e-16) + b2_ref[...]
    mx = jnp.max(lg, axis=1, keepdims=True)
    s = lg - mx
    o_ref[...] = s - jnp.log(jnp.sum(jnp.exp(s), axis=1, keepdims=True))


def _final(md2, b2r):
    blk = 512
    nb = N_PAD // blk
    return pl.pallas_call(
        _final_body,
        grid=(nb,),
        in_specs=[
            pl.BlockSpec((blk, C2P), lambda i: (i, 0)),
            pl.BlockSpec((blk, C2P), lambda i: (i + nb, 0)),
            pl.BlockSpec((1, C2), lambda i: (0, 0)),
        ],
        out_specs=pl.BlockSpec((blk, C2), lambda i: (i, 0)),
        out_shape=jax.ShapeDtypeStruct((N_PAD, C2), _F32),
    )(md2, md2, b2r)


def _l1_body(src_hbm, dst_hbm, zer_hbm, as_hbm, ad_hbm, h_hbm,
             msg_hbm,
             src0, dst0, dc0, gs0, gd0, hb0, mb0, si0, sg0, sc0,
             src1, dst1, dc1, gs1, gd1, hb1, mb1, si1, sg1, sc1,
             acc_m):
    cid = lax.axis_index("c")
    sid = lax.axis_index("s")
    wid = cid * NS + sid

    @pl.when(sid < NS - 1)
    def _():
        pltpu.sync_copy(zer_hbm.at[pl.ds(0, RPA)],
                        acc_m.at[pl.ds(sid * RPA, RPA)])

    @pl.when(sid == NS - 1)
    def _():
        pltpu.sync_copy(zer_hbm.at[pl.ds(0, RPA_LAST)],
                        acc_m.at[pl.ds(15 * RPA, RPA_LAST)])

    plsc.subcore_barrier()

    B0 = (src0, dst0, dc0, gs0, gd0, hb0, mb0, si0, sg0, sc0)
    B1 = (src1, dst1, dc1, gs1, gd1, hb1, mb1, si1, sg1, sc1)

    def idx_fetch(bs, g):
        src_v, dst_v, si = bs[0], bs[1], bs[7]
        base = wid * EPW1 + g * K1
        pltpu.async_copy(src_hbm.at[pl.ds(base, K1)], src_v, si)
        pltpu.async_copy(dst_hbm.at[pl.ds(base, K1)], dst_v, si)

    def wait_idx(bs, g):
        src_v, dst_v, si = bs[0], bs[1], bs[7]
        base = wid * EPW1 + g * K1
        pltpu.make_async_copy(src_hbm.at[pl.ds(base, K1)], src_v, si).wait()
        pltpu.make_async_copy(dst_hbm.at[pl.ds(base, K1)], dst_v, si).wait()

    def gathers(bs):
        src_v, dst_v, gs, gd, hb, sg = bs[0], bs[1], bs[3], bs[4], bs[5], bs[8]
        pltpu.async_copy(as_hbm.at[src_v], gs, sg)
        pltpu.async_copy(ad_hbm.at[dst_v], gd, sg)
        pltpu.async_copy(h_hbm.at[src_v], hb, sg)

    def wait_gathers(bs):
        src_v, dst_v, gs, gd, hb, sg = bs[0], bs[1], bs[3], bs[4], bs[5], bs[8]
        pltpu.make_async_copy(as_hbm.at[src_v], gs, sg).wait()
        pltpu.make_async_copy(ad_hbm.at[dst_v], gd, sg).wait()
        pltpu.make_async_copy(h_hbm.at[src_v], hb, sg).wait()

    def wait_out(bs):
        dstc, mb, sc = bs[2], bs[6], bs[9]
        pltpu.make_async_copy(mb, acc_m.at[dstc], sc).wait()

    def snap(bs):
        dst_v, dstc = bs[1], bs[2]
        for q in range(K1 // 16):
            dstc[pl.ds(q * 16, 16)] = dst_v[pl.ds(q * 16, 16)]

    def proc(bs):
        dstc, gs, gd, hb, mb = bs[2], bs[3], bs[4], bs[5], bs[6]
        sc = bs[9]

        def erow(r, c):
            for j in range(16):
                k = r * 16 + j
                v = gs[k, pl.ds(0, 16)] + gd[k, pl.ds(0, 16)]
                v = jnp.maximum(v, 0.2 * v)
                v = jnp.exp(v)
                mb[k, pl.ds(128, 16)] = v
                for h in range(H1):
                    mb[k, pl.ds(h * 16, 16)] = hb[k, pl.ds(h * 16, 16)] * v[h]
            return c

        lax.fori_loop(0, K1 // 16, erow, 0)
        pltpu.async_copy(mb, acc_m.at[dstc], sc, add=True)

    idx_fetch(B0, 0)
    idx_fetch(B1, 1)
    wait_idx(B0, 0)
    gathers(B0)

    def pair(i, carry):
        g = 2 * i
        wait_idx(B1, g + 1)
        gathers(B1)
        wait_gathers(B0)

        @pl.when(g >= 2)
        def _():
            wait_out(B0)

        snap(B0)
        idx_fetch(B0, g + 2)
        proc(B0)
        wait_idx(B0, g + 2)
        gathers(B0)
        wait_gathers(B1)

        @pl.when(g >= 2)
        def _():
            wait_out(B1)

        snap(B1)
        idx_fetch(B1, g + 3)
        proc(B1)
        return carry

    lax.fori_loop(0, (CH1 - 1) // 2, pair, 0)
    wait_gathers(B0)
    wait_out(B0)
    snap(B0)
    proc(B0)
    wait_idx(B1, CH1)
    wait_out(B1)
    wait_out(B0)
    plsc.subcore_barrier()

    @pl.when(sid < NS - 1)
    def _():
        pltpu.sync_copy(acc_m.at[pl.ds(sid * RPA, RPA)],
                        msg_hbm.at[pl.ds(cid * N_PAD + sid * RPA, RPA)])

    @pl.when(sid == NS - 1)
    def _():
        pltpu.sync_copy(acc_m.at[pl.ds(15 * RPA, RPA_LAST)],
                        msg_hbm.at[pl.ds(cid * N_PAD + 15 * RPA, RPA_LAST)])


def _l1_call():
    mesh = plsc.VectorSubcoreMesh(core_axis_name="c", subcore_axis_name="s")
    buf = [
        pltpu.VMEM((K1,), jnp.int32),
        pltpu.VMEM((K1,), jnp.int32),
        pltpu.VMEM((K1,), jnp.int32),
        pltpu.VMEM((K1, 16), _F32),
        pltpu.VMEM((K1, 16), _F32),
        pltpu.VMEM((K1, D), _F32),
        pltpu.VMEM((K1, R1), _F32),
        pltpu.SemaphoreType.DMA,
        pltpu.SemaphoreType.DMA,
        pltpu.SemaphoreType.DMA,
    ]
    return pl.kernel(
        _l1_body,
        out_type=jax.ShapeDtypeStruct((NC * N_PAD, R1), _F32),
        mesh=mesh,
        scratch_types=buf + buf + [
            pltpu.VMEM_SHARED((N_ACC, R1), _F32),
        ],
        compiler_params=_SC_PARAMS,
    )


def _l2_body(src_hbm, dst_hbm, zer48_hbm, a2d_hbm, h2_hbm, out_hbm,
             src0, dst0, dc0, gd0, hb0, mb0, si0, sg0, sc0,
             src1, dst1, dc1, gd1, hb1, mb1, si1, sg1, sc1, accum):
    cid = lax.axis_index("c")
    sid = lax.axis_index("s")
    wid = cid * NS + sid
    pltpu.sync_copy(zer48_hbm, accum.at[pl.ds(sid * RPT, RPT)])
    plsc.subcore_barrier()
    iot = lax.iota(jnp.int32, 16)

    B0 = (src0, dst0, dc0, gd0, hb0, mb0, si0, sg0, sc0)
    B1 = (src1, dst1, dc1, gd1, hb1, mb1, si1, sg1, sc1)

    def idx_fetch(bs, g):
        src_v, dst_v, si = bs[0], bs[1], bs[6]
        base = wid * EPW2 + g * K2
        pltpu.async_copy(src_hbm.at[pl.ds(base, K2)], src_v, si)
        pltpu.async_copy(dst_hbm.at[pl.ds(base, K2)], dst_v, si)

    def wait_idx(bs, g):
        src_v, dst_v, si = bs[0], bs[1], bs[6]
        base = wid * EPW2 + g * K2
        pltpu.make_async_copy(src_hbm.at[pl.ds(base, K2)], src_v, si).wait()
        pltpu.make_async_copy(dst_hbm.at[pl.ds(base, K2)], dst_v, si).wait()

    def gathers(bs):
        src_v, dst_v, gd, hb, sg = bs[0], bs[1], bs[3], bs[4], bs[7]
        pltpu.async_copy(a2d_hbm.at[dst_v], gd, sg)
        pltpu.async_copy(h2_hbm.at[src_v], hb, sg)

    def wait_gathers(bs):
        src_v, dst_v, gd, hb, sg = bs[0], bs[1], bs[3], bs[4], bs[7]
        pltpu.make_async_copy(a2d_hbm.at[dst_v], gd, sg).wait()
        pltpu.make_async_copy(h2_hbm.at[src_v], hb, sg).wait()

    def wait_out(bs):
        dstc, mb, sc = bs[2], bs[5], bs[8]
        pltpu.make_async_copy(mb, accum.at[dstc], sc).wait()

    def snap(bs):
        dst_v, dstc = bs[1], bs[2]
        for q in range(K2 // 16):
            dstc[pl.ds(q * 16, 16)] = dst_v[pl.ds(q * 16, 16)]

    def proc(bs):
        dstc, gd, hb, mb, sc = bs[2], bs[3], bs[4], bs[5], bs[8]

        def mrow(r, c):
            for j in range(16):
                k = r * 16 + j
                h2v2 = hb[k, pl.ds(32, 16)]
                vw = gd[k, pl.ds(0, 16)] + h2v2[8]
                vw = jnp.maximum(vw, 0.2 * vw)
                vw = jnp.exp(vw)
                ws = vw[0]
                mb[k, pl.ds(0, 16)] = hb[k, pl.ds(0, 16)] * ws
                mb[k, pl.ds(16, 16)] = hb[k, pl.ds(16, 16)] * ws
                v2 = h2v2 * ws
                v2 = jnp.where(iot == 8, ws, v2)
                mb[k, pl.ds(32, 16)] = v2
            return c

        lax.fori_loop(0, K2 // 16, mrow, 0)
        pltpu.async_copy(mb, accum.at[dstc], sc, add=True)

    idx_fetch(B0, 0)
    idx_fetch(B1, 1)
    wait_idx(B0, 0)
    gathers(B0)

    def pair(i, carry):
        g = 2 * i
        wait_idx(B1, g + 1)
        gathers(B1)
        wait_gathers(B0)

        @pl.when(g >= 2)
        def _():
            wait_out(B0)

        snap(B0)
        idx_fetch(B0, g + 2)
        proc(B0)
        wait_idx(B0, g + 2)
        gathers(B0)
        wait_gathers(B1)

        @pl.when(g >= 2)
        def _():
            wait_out(B1)

        snap(B1)
        idx_fetch(B1, g + 3)
        proc(B1)
        return carry

    lax.fori_loop(0, (CH2 - 1) // 2, pair, 0)
    wait_gathers(B0)
    wait_out(B0)
    snap(B0)
    proc(B0)
    wait_idx(B1, CH2)
    wait_out(B1)
    wait_out(B0)
    plsc.subcore_barrier()
    row0 = cid * N_PAD + sid * RPT
    pltpu.sync_copy(accum.at[pl.ds(sid * RPT, RPT)],
                    out_hbm.at[pl.ds(row0, RPT)])


def _l2_call():
    mesh = plsc.VectorSubcoreMesh(core_axis_name="c", subcore_axis_name="s")
    buf = [
        pltpu.VMEM((K2,), jnp.int32),
        pltpu.VMEM((K2,), jnp.int32),
        pltpu.VMEM((K2,), jnp.int32),
        pltpu.VMEM((K2, 16), _F32),
        pltpu.VMEM((K2, C2P), _F32),
        pltpu.VMEM((K2, C2P), _F32),
        pltpu.SemaphoreType.DMA,
        pltpu.SemaphoreType.DMA,
        pltpu.SemaphoreType.DMA,
    ]
    return pl.kernel(
        _l2_body,
        out_type=jax.ShapeDtypeStruct((NC * N_PAD, C2P), _F32),
        mesh=mesh,
        scratch_types=buf + buf + [
            pltpu.VMEM_SHARED((N_PAD, C2P), _F32),
        ],
        compiler_params=_SC_PARAMS,
    )


def kernel(x, edge_index, W1, att_src1, att_dst1, b1, W2, att_src2, att_dst2, b2):
    xp = jnp.pad(x.astype(_F32), ((0, N_PAD - N), (0, 0)))
    loop = jnp.arange(N, dtype=jnp.int32)
    pad = jnp.full((E_ALL - E_TOT,), N, jnp.int32)
    src = jnp.concatenate([edge_index[0].astype(jnp.int32), loop, pad])
    dst = jnp.concatenate([edge_index[1].astype(jnp.int32), loop, pad])

    eye8 = jnp.eye(H1, dtype=_F32)
    As = jnp.pad(
        jnp.einsum("hc,hg->hcg", att_src1[0].astype(_F32), eye8).reshape(D, H1),
        ((0, 0), (0, 8)))
    Ad = jnp.pad(
        jnp.einsum("hc,hg->hcg", att_dst1[0].astype(_F32), eye8).reshape(D, H1),
        ((0, 0), (0, 8)))
    E8 = jnp.repeat(eye8, 16, axis=1)
    vs2 = W2.astype(_F32) @ att_src2[0, 0].astype(_F32)
    vd2 = W2.astype(_F32) @ att_dst2[0, 0].astype(_F32)
    W2e = jnp.concatenate(
        [W2.astype(_F32), vs2[:, None], vd2[:, None],
         jnp.zeros((D, C2P - C2 - 2), _F32)], axis=1)
    zer = jnp.zeros((RPT, R1), _F32)
    zer48 = jnp.zeros((RPT, C2P), _F32)

    h1, ast, adt = _node1(xp, W1.astype(_F32), As, Ad)
    msg1 = _l1_call()(src, dst, zer, ast, adt, h1)
    h2e, a2d = _node2(msg1, E8, b1.astype(_F32).reshape(1, D), W2e)
    md2 = _l2_call()(src, dst, zer48, a2d, h2e)
    out = _final(md2, b2.astype(_F32).reshape(1, C2))
    return out[:N]


# speedup vs baseline: 1.2943x; 1.2943x over previous
import jax
import jax.numpy as jnp
from jax import lax
from jax.experimental import pallas as pl
from jax.experimental.pallas import tpu as pltpu
from jax.experimental.pallas import tpu_sc as plsc

N = 10000
N_PAD = 10240
E_TOT = 320000 + N
D = 128
H1 = 8
C2 = 40
C2P = 48

NC = 2
NS = 16
NW = NC * NS
RPT = N_PAD // NS
N_ACC = 10008
RPA = 632
RPA_LAST = N_ACC - 15 * RPA

K1 = 64
CH1 = 163
EPW1 = K1 * CH1
E_PAD = NW * EPW1
E_ALL = E_PAD + K1

K2 = 128
CH2 = 81
EPW2 = K2 * CH2

_F32 = jnp.float32
_SC_PARAMS = pltpu.CompilerParams(use_tc_tiling_on_sc=False)


def _node1_body(x_ref, w_ref, as_ref, ad_ref, h_ref, s_ref, d_ref):
    h = jnp.dot(x_ref[...], w_ref[...], preferred_element_type=_F32)
    h_ref[...] = h
    s_ref[...] = jnp.dot(h, as_ref[...], preferred_element_type=_F32)
    d_ref[...] = jnp.dot(h, ad_ref[...], preferred_element_type=_F32)


def _node1(xp, W1, As, Ad):
    blk = 512
    return pl.pallas_call(
        _node1_body,
        grid=(N_PAD // blk,),
        in_specs=[
            pl.BlockSpec((blk, D), lambda i: (i, 0)),
            pl.BlockSpec((D, D), lambda i: (0, 0)),
            pl.BlockSpec((D, 16), lambda i: (0, 0)),
            pl.BlockSpec((D, 16), lambda i: (0, 0)),
        ],
        out_specs=[
            pl.BlockSpec((blk, D), lambda i: (i, 0)),
            pl.BlockSpec((blk, 16), lambda i: (i, 0)),
            pl.BlockSpec((blk, 16), lambda i: (i, 0)),
        ],
        out_shape=[
            jax.ShapeDtypeStruct((N_PAD, D), _F32),
            jax.ShapeDtypeStruct((N_PAD, 16), _F32),
            jax.ShapeDtypeStruct((N_PAD, 16), _F32),
        ],
    )(xp, W1, As, Ad)


def _node2_body(m0_ref, m1_ref, d0_ref, d1_ref, e8_ref, b1_ref, w2_ref,
                h2_ref, d_ref):
    m = m0_ref[...] + m1_ref[...]
    d8 = d0_ref[...][:, 0:8] + d1_ref[...][:, 0:8]
    den = jnp.dot(d8, e8_ref[...], preferred_element_type=_F32)
    hr = jnp.maximum(m / (den + 1e-16) + b1_ref[...], 0.0)
    h2 = jnp.dot(hr, w2_ref[...], preferred_element_type=_F32)
    h2_ref[...] = h2
    d_ref[...] = jnp.broadcast_to(h2[:, 41:42], (h2.shape[0], 16))


def _node2(msg, den, E8, b1r, W2e):
    blk = 512
    nb = N_PAD // blk
    return pl.pallas_call(
        _node2_body,
        grid=(nb,),
        in_specs=[
            pl.BlockSpec((blk, D), lambda i: (i, 0)),
            pl.BlockSpec((blk, D), lambda i: (i + nb, 0)),
            pl.BlockSpec((blk, 16), lambda i: (i, 0)),
            pl.BlockSpec((blk, 16), lambda i: (i + nb, 0)),
            pl.BlockSpec((8, D), lambda i: (0, 0)),
            pl.BlockSpec((1, D), lambda i: (0, 0)),
            pl.BlockSpec((D, C2P), lambda i: (0, 0)),
        ],
        out_specs=[
            pl.BlockSpec((blk, C2P), lambda i: (i, 0)),
            pl.BlockSpec((blk, 16), lambda i: (i, 0)),
        ],
        out_shape=[
            jax.ShapeDtypeStruct((N_PAD, C2P), _F32),
            jax.ShapeDtypeStruct((N_PAD, 16), _F32),
        ],
    )(msg, msg, den, den, E8, b1r, W2e)


def _final_body(m0_ref, m1_ref, b2_ref, o_ref):
    m = m0_ref[...] + m1_ref[...]
    lg = m[:, 0:40] / (m[:, 40:41] + 1e-16) + b2_ref[...]
    mx = jnp.max(lg, axis=1, keepdims=True)
    s = lg - mx
    o_ref[...] = s - jnp.log(jnp.sum(jnp.exp(s), axis=1, keepdims=True))


def _final(md2, b2r):
    blk = 512
    nb = N_PAD // blk
    return pl.pallas_call(
        _final_body,
        grid=(nb,),
        in_specs=[
            pl.BlockSpec((blk, C2P), lambda i: (i, 0)),
            pl.BlockSpec((blk, C2P), lambda i: (i + nb, 0)),
            pl.BlockSpec((1, C2), lambda i: (0, 0)),
        ],
        out_specs=pl.BlockSpec((blk, C2), lambda i: (i, 0)),
        out_shape=jax.ShapeDtypeStruct((N_PAD, C2), _F32),
    )(md2, md2, b2r)


def _l1_body(src_hbm, dst_hbm, zer_hbm, zer16_hbm, as_hbm, ad_hbm, h_hbm,
             msg_hbm, den_hbm,
             src0, dst0, dc0, gs0, gd0, hb0, mb0, db0, si0, sg0, sc0, sd0,
             src1, dst1, dc1, gs1, gd1, hb1, mb1, db1, si1, sg1, sc1, sd1,
             acc_m, acc_d):
    cid = lax.axis_index("c")
    sid = lax.axis_index("s")
    wid = cid * NS + sid

    @pl.when(sid < NS - 1)
    def _():
        pltpu.sync_copy(zer_hbm.at[pl.ds(0, RPA)],
                        acc_m.at[pl.ds(sid * RPA, RPA)])
        pltpu.sync_copy(zer16_hbm.at[pl.ds(0, RPA)],
                        acc_d.at[pl.ds(sid * RPA, RPA)])

    @pl.when(sid == NS - 1)
    def _():
        pltpu.sync_copy(zer_hbm.at[pl.ds(0, RPA_LAST)],
                        acc_m.at[pl.ds(15 * RPA, RPA_LAST)])
        pltpu.sync_copy(zer16_hbm.at[pl.ds(0, RPA_LAST)],
                        acc_d.at[pl.ds(15 * RPA, RPA_LAST)])

    plsc.subcore_barrier()

    B0 = (src0, dst0, dc0, gs0, gd0, hb0, mb0, db0, si0, sg0, sc0, sd0)
    B1 = (src1, dst1, dc1, gs1, gd1, hb1, mb1, db1, si1, sg1, sc1, sd1)

    def idx_fetch(bs, g):
        src_v, dst_v, si = bs[0], bs[1], bs[8]
        base = wid * EPW1 + g * K1
        pltpu.async_copy(src_hbm.at[pl.ds(base, K1)], src_v, si)
        pltpu.async_copy(dst_hbm.at[pl.ds(base, K1)], dst_v, si)

    def wait_idx(bs, g):
        src_v, dst_v, si = bs[0], bs[1], bs[8]
        base = wid * EPW1 + g * K1
        pltpu.make_async_copy(src_hbm.at[pl.ds(base, K1)], src_v, si).wait()
        pltpu.make_async_copy(dst_hbm.at[pl.ds(base, K1)], dst_v, si).wait()

    def gathers(bs):
        src_v, dst_v, gs, gd, hb, sg = bs[0], bs[1], bs[3], bs[4], bs[5], bs[9]
        pltpu.async_copy(as_hbm.at[src_v], gs, sg)
        pltpu.async_copy(ad_hbm.at[dst_v], gd, sg)
        pltpu.async_copy(h_hbm.at[src_v], hb, sg)

    def wait_gathers(bs):
        src_v, dst_v, gs, gd, hb, sg = bs[0], bs[1], bs[3], bs[4], bs[5], bs[9]
        pltpu.make_async_copy(as_hbm.at[src_v], gs, sg).wait()
        pltpu.make_async_copy(ad_hbm.at[dst_v], gd, sg).wait()
        pltpu.make_async_copy(h_hbm.at[src_v], hb, sg).wait()

    def wait_out(bs):
        dstc, mb, db, sc, sd = bs[2], bs[6], bs[7], bs[10], bs[11]
        pltpu.make_async_copy(mb, acc_m.at[dstc], sc).wait()
        pltpu.make_async_copy(db, acc_d.at[dstc], sd).wait()

    def snap(bs):
        dst_v, dstc = bs[1], bs[2]
        for q in range(K1 // 16):
            dstc[pl.ds(q * 16, 16)] = dst_v[pl.ds(q * 16, 16)]

    def proc(bs):
        dstc, gs, gd, hb, mb, db = bs[2], bs[3], bs[4], bs[5], bs[6], bs[7]
        sc, sd = bs[10], bs[11]

        def erow(r, c):
            for j in range(16):
                k = r * 16 + j
                v = gs[k, pl.ds(0, 16)] + gd[k, pl.ds(0, 16)]
                v = jnp.maximum(v, 0.2 * v)
                v = jnp.exp(v)
                db[k, pl.ds(0, 16)] = v
                for h in range(H1):
                    mb[k, pl.ds(h * 16, 16)] = hb[k, pl.ds(h * 16, 16)] * v[h]
            return c

        lax.fori_loop(0, K1 // 16, erow, 0)
        pltpu.async_copy(mb, acc_m.at[dstc], sc, add=True)
        pltpu.async_copy(db, acc_d.at[dstc], sd, add=True)

    idx_fetch(B0, 0)
    idx_fetch(B1, 1)
    wait_idx(B0, 0)
    gathers(B0)

    def pair(i, carry):
        g = 2 * i
        wait_idx(B1, g + 1)
        gathers(B1)
        wait_gathers(B0)

        @pl.when(g >= 2)
        def _():
            wait_out(B0)

        snap(B0)
        idx_fetch(B0, g + 2)
        proc(B0)
        wait_idx(B0, g + 2)
        gathers(B0)
        wait_gathers(B1)

        @pl.when(g >= 2)
        def _():
            wait_out(B1)

        snap(B1)
        idx_fetch(B1, g + 3)
        proc(B1)
        return carry

    lax.fori_loop(0, (CH1 - 1) // 2, pair, 0)
    wait_gathers(B0)
    wait_out(B0)
    snap(B0)
    proc(B0)
    wait_idx(B1, CH1)
    wait_out(B1)
    wait_out(B0)
    plsc.subcore_barrier()

    @pl.when(sid < NS - 1)
    def _():
        pltpu.sync_copy(acc_m.at[pl.ds(sid * RPA, RPA)],
                        msg_hbm.at[pl.ds(cid * N_PAD + sid * RPA, RPA)])
        pltpu.sync_copy(acc_d.at[pl.ds(sid * RPA, RPA)],
                        den_hbm.at[pl.ds(cid * N_PAD + sid * RPA, RPA)])

    @pl.when(sid == NS - 1)
    def _():
        pltpu.sync_copy(acc_m.at[pl.ds(15 * RPA, RPA_LAST)],
                        msg_hbm.at[pl.ds(cid * N_PAD + 15 * RPA, RPA_LAST)])
        pltpu.sync_copy(acc_d.at[pl.ds(15 * RPA, RPA_LAST)],
                        den_hbm.at[pl.ds(cid * N_PAD + 15 * RPA, RPA_LAST)])


def _l1_call():
    mesh = plsc.VectorSubcoreMesh(core_axis_name="c", subcore_axis_name="s")
    buf = [
        pltpu.VMEM((K1,), jnp.int32),
        pltpu.VMEM((K1,), jnp.int32),
        pltpu.VMEM((K1,), jnp.int32),
        pltpu.VMEM((K1, 16), _F32),
        pltpu.VMEM((K1, 16), _F32),
        pltpu.VMEM((K1, D), _F32),
        pltpu.VMEM((K1, D), _F32),
        pltpu.VMEM((K1, 16), _F32),
        pltpu.SemaphoreType.DMA,
        pltpu.SemaphoreType.DMA,
        pltpu.SemaphoreType.DMA,
        pltpu.SemaphoreType.DMA,
    ]
    return pl.kernel(
        _l1_body,
        out_type=[
            jax.ShapeDtypeStruct((NC * N_PAD, D), _F32),
            jax.ShapeDtypeStruct((NC * N_PAD, 16), _F32),
        ],
        mesh=mesh,
        scratch_types=buf + buf + [
            pltpu.VMEM_SHARED((N_ACC, D), _F32),
            pltpu.VMEM_SHARED((N_ACC, 16), _F32),
        ],
        compiler_params=_SC_PARAMS,
    )


def _l2_body(src_hbm, dst_hbm, zer48_hbm, a2d_hbm, h2_hbm, out_hbm,
             src0, dst0, dc0, gd0, hb0, mb0, si0, sg0, sc0,
             src1, dst1, dc1, gd1, hb1, mb1, si1, sg1, sc1, accum):
    cid = lax.axis_index("c")
    sid = lax.axis_index("s")
    wid = cid * NS + sid
    pltpu.sync_copy(zer48_hbm, accum.at[pl.ds(sid * RPT, RPT)])
    plsc.subcore_barrier()
    iot = lax.iota(jnp.int32, 16)

    B0 = (src0, dst0, dc0, gd0, hb0, mb0, si0, sg0, sc0)
    B1 = (src1, dst1, dc1, gd1, hb1, mb1, si1, sg1, sc1)

    def idx_fetch(bs, g):
        src_v, dst_v, si = bs[0], bs[1], bs[6]
        base = wid * EPW2 + g * K2
        pltpu.async_copy(src_hbm.at[pl.ds(base, K2)], src_v, si)
        pltpu.async_copy(dst_hbm.at[pl.ds(base, K2)], dst_v, si)

    def wait_idx(bs, g):
        src_v, dst_v, si = bs[0], bs[1], bs[6]
        base = wid * EPW2 + g * K2
        pltpu.make_async_copy(src_hbm.at[pl.ds(base, K2)], src_v, si).wait()
        pltpu.make_async_copy(dst_hbm.at[pl.ds(base, K2)], dst_v, si).wait()

    def gathers(bs):
        src_v, dst_v, gd, hb, sg = bs[0], bs[1], bs[3], bs[4], bs[7]
        pltpu.async_copy(a2d_hbm.at[dst_v], gd, sg)
        pltpu.async_copy(h2_hbm.at[src_v], hb, sg)

    def wait_gathers(bs):
        src_v, dst_v, gd, hb, sg = bs[0], bs[1], bs[3], bs[4], bs[7]
        pltpu.make_async_copy(a2d_hbm.at[dst_v], gd, sg).wait()
        pltpu.make_async_copy(h2_hbm.at[src_v], hb, sg).wait()

    def wait_out(bs):
        dstc, mb, sc = bs[2], bs[5], bs[8]
        pltpu.make_async_copy(mb, accum.at[dstc], sc).wait()

    def snap(bs):
        dst_v, dstc = bs[1], bs[2]
        for q in range(K2 // 16):
            dstc[pl.ds(q * 16, 16)] = dst_v[pl.ds(q * 16, 16)]

    def proc(bs):
        dstc, gd, hb, mb, sc = bs[2], bs[3], bs[4], bs[5], bs[8]

        def mrow(r, c):
            for j in range(16):
                k = r * 16 + j
                h2v2 = hb[k, pl.ds(32, 16)]
                vw = gd[k, pl.ds(0, 16)] + h2v2[8]
                vw = jnp.maximum(vw, 0.2 * vw)
                vw = jnp.exp(vw)
                ws = vw[0]
                mb[k, pl.ds(0, 16)] = hb[k, pl.ds(0, 16)] * ws
                mb[k, pl.ds(16, 16)] = hb[k, pl.ds(16, 16)] * ws
                v2 = h2v2 * ws
                v2 = jnp.where(iot == 8, ws, v2)
                mb[k, pl.ds(32, 16)] = v2
            return c

        lax.fori_loop(0, K2 // 16, mrow, 0)
        pltpu.async_copy(mb, accum.at[dstc], sc, add=True)

    idx_fetch(B0, 0)
    idx_fetch(B1, 1)
    wait_idx(B0, 0)
    gathers(B0)

    def pair(i, carry):
        g = 2 * i
        wait_idx(B1, g + 1)
        gathers(B1)
        wait_gathers(B0)

        @pl.when(g >= 2)
        def _():
            wait_out(B0)

        snap(B0)
        idx_fetch(B0, g + 2)
        proc(B0)
        wait_idx(B0, g + 2)
        gathers(B0)
        wait_gathers(B1)

        @pl.when(g >= 2)
        def _():
            wait_out(B1)

        snap(B1)
        idx_fetch(B1, g + 3)
        proc(B1)
        return carry

    lax.fori_loop(0, (CH2 - 1) // 2, pair, 0)
    wait_gathers(B0)
    wait_out(B0)
    snap(B0)
    proc(B0)
    wait_idx(B1, CH2)
    wait_out(B1)
    wait_out(B0)
    plsc.subcore_barrier()
    row0 = cid * N_PAD + sid * RPT
    pltpu.sync_copy(accum.at[pl.ds(sid * RPT, RPT)],
                    out_hbm.at[pl.ds(row0, RPT)])


def _l2_call():
    mesh = plsc.VectorSubcoreMesh(core_axis_name="c", subcore_axis_name="s")
    buf = [
        pltpu.VMEM((K2,), jnp.int32),
        pltpu.VMEM((K2,), jnp.int32),
        pltpu.VMEM((K2,), jnp.int32),
        pltpu.VMEM((K2, 16), _F32),
        pltpu.VMEM((K2, C2P), _F32),
        pltpu.VMEM((K2, C2P), _F32),
        pltpu.SemaphoreType.DMA,
        pltpu.SemaphoreType.DMA,
        pltpu.SemaphoreType.DMA,
    ]
    return pl.kernel(
        _l2_body,
        out_type=jax.ShapeDtypeStruct((NC * N_PAD, C2P), _F32),
        mesh=mesh,
        scratch_types=buf + buf + [
            pltpu.VMEM_SHARED((N_PAD, C2P), _F32),
        ],
        compiler_params=_SC_PARAMS,
    )


def kernel(x, edge_index, W1, att_src1, att_dst1, b1, W2, att_src2, att_dst2, b2):
    xp = jnp.pad(x.astype(_F32), ((0, N_PAD - N), (0, 0)))
    loop = jnp.arange(N, dtype=jnp.int32)
    pad = jnp.full((E_ALL - E_TOT,), N, jnp.int32)
    src = jnp.concatenate([edge_index[0].astype(jnp.int32), loop, pad])
    dst = jnp.concatenate([edge_index[1].astype(jnp.int32), loop, pad])

    eye8 = jnp.eye(H1, dtype=_F32)
    As = jnp.pad(
        jnp.einsum("hc,hg->hcg", att_src1[0].astype(_F32), eye8).reshape(D, H1),
        ((0, 0), (0, 8)))
    Ad = jnp.pad(
        jnp.einsum("hc,hg->hcg", att_dst1[0].astype(_F32), eye8).reshape(D, H1),
        ((0, 0), (0, 8)))
    E8 = jnp.repeat(eye8, 16, axis=1)
    vs2 = W2.astype(_F32) @ att_src2[0, 0].astype(_F32)
    vd2 = W2.astype(_F32) @ att_dst2[0, 0].astype(_F32)
    W2e = jnp.concatenate(
        [W2.astype(_F32), vs2[:, None], vd2[:, None],
         jnp.zeros((D, C2P - C2 - 2), _F32)], axis=1)
    zer = jnp.zeros((RPT, D), _F32)
    zer16 = jnp.zeros((RPT, 16), _F32)
    zer48 = jnp.zeros((RPT, C2P), _F32)

    h1, ast, adt = _node1(xp, W1.astype(_F32), As, Ad)
    msg1, den1 = _l1_call()(src, dst, zer, zer16, ast, adt, h1)
    h2e, a2d = _node2(msg1, den1, E8, b1.astype(_F32).reshape(1, D), W2e)
    md2 = _l2_call()(src, dst, zer48, a2d, h2e)
    out = _final(md2, b2.astype(_F32).reshape(1, C2))
    return out[:N]
